# baseline (speedup 1.0000x reference)
import jax
import jax.numpy as jnp
from jax import lax
from jax.experimental import pallas as pl
from jax.experimental.pallas import tpu as pltpu
from jax.experimental.pallas import tpu_sc as plsc

N = 50000
E = 1600000
DIN = 128
DOUT = 32
R = 3
L = 3

NC, NS = 2, 16
NW = NC * NS
NBLK = 204
BR = 2
BLK = BR * 128
RW = NBLK * BR
IR = NW * RW
EP = IR * 128
CCH = 32
CNCH = (IR // NC) // CCH
ACC = 50176
TPT = ACC // NS
CNT = 4 * ACC
CPT = CNT // NS
BN = 2000

_f32 = jnp.float32


def _idx_body(src_ref, et_ref, dst_ref, pk_ref, didx_ref):
    et = et_ref[...]
    pk_ref[:, 0, :] = src_ref[...] * 4 + et
    pk_ref[:, 1, :] = dst_ref[...]
    didx_ref[...] = dst_ref[...] * 4 + et


def _make_idx():
    blk = pl.BlockSpec((544, 128), lambda i: (i, 0))
    return pl.pallas_call(
        _idx_body,
        grid=(IR // 544,),
        in_specs=[blk, blk, blk],
        out_specs=[pl.BlockSpec((544, 2, 128), lambda i: (i, 0, 0)), blk],
        out_shape=[jax.ShapeDtypeStruct((IR, 2, 128), jnp.int32),
                   jax.ShapeDtypeStruct((IR, 128), jnp.int32)],
    )


def _inv_body(cnt_ref, inv_ref):
    c = cnt_ref[0] + cnt_ref[1]
    inv_ref[...] = 1.0 / jnp.maximum(c, 1.0)


def _make_inv():
    return pl.pallas_call(
        _inv_body,
        in_specs=[pl.BlockSpec((2, CNT // 128, 128), lambda: (0, 0, 0))],
        out_specs=pl.BlockSpec((CNT // 128, 128), lambda: (0, 0)),
        out_shape=jax.ShapeDtypeStruct((CNT // 128, 128), _f32),
    )


def _pre_body(x_ref, wf_ref, bf_ref, wa_ref, h4_ref):
    h0 = jnp.dot(x_ref[...], wf_ref[...], preferred_element_type=_f32)
    h0 = h0 + bf_ref[...]
    h4_ref[...] = jnp.dot(h0, wa_ref[...], preferred_element_type=_f32)


def _make_pre():
    return pl.pallas_call(
        _pre_body,
        grid=(N // BN,),
        in_specs=[
            pl.BlockSpec((BN, DIN), lambda i: (i, 0)),
            pl.BlockSpec((DIN, DOUT), lambda i: (0, 0)),
            pl.BlockSpec((1, DOUT), lambda i: (0, 0)),
            pl.BlockSpec((DOUT, 4 * DOUT), lambda i: (0, 0)),
        ],
        out_specs=pl.BlockSpec((BN, 4 * DOUT), lambda i: (i, 0)),
        out_shape=jax.ShapeDtypeStruct((N, 4 * DOUT), _f32),
    )


def _ln_core(p_ref, root_ref, bc_ref, g_ref, b_ref):
    y = p_ref[0] + p_ref[1] + root_ref[...][:, 96:128] + bc_ref[...]
    y = jnp.maximum(y, 0.0)
    mu = jnp.mean(y, axis=-1, keepdims=True)
    d = y - mu
    var = jnp.mean(d * d, axis=-1, keepdims=True)
    return d * lax.rsqrt(var + 1e-5) * g_ref[...] + b_ref[...]


def _ln_mid_body(p_ref, root_ref, bc_ref, g_ref, b_ref, wn_ref, out_ref):
    yn = _ln_core(p_ref, root_ref, bc_ref, g_ref, b_ref)
    out_ref[...] = jnp.dot(yn, wn_ref[...], preferred_element_type=_f32)


def _ln_last_body(p_ref, root_ref, bc_ref, g_ref, b_ref, out_ref):
    out_ref[...] = _ln_core(p_ref, root_ref, bc_ref, g_ref, b_ref)


def _make_ln(last):
    vec = pl.BlockSpec((1, DOUT), lambda i: (0, 0))
    in_specs = [
        pl.BlockSpec((2, BN, DOUT), lambda i: (0, i, 0)),
        pl.BlockSpec((BN, 4 * DOUT), lambda i: (i, 0)),
        vec, vec, vec,
    ]
    if last:
        return pl.pallas_call(
            _ln_last_body,
            grid=(N // BN,),
            in_specs=in_specs,
            out_specs=pl.BlockSpec((BN, DOUT), lambda i: (i, 0)),
            out_shape=jax.ShapeDtypeStruct((N, DOUT), _f32),
        )
    in_specs = in_specs + [pl.BlockSpec((DOUT, 4 * DOUT), lambda i: (0, 0))]
    return pl.pallas_call(
        _ln_mid_body,
        grid=(N // BN,),
        in_specs=in_specs,
        out_specs=pl.BlockSpec((BN, 4 * DOUT), lambda i: (i, 0)),
        out_shape=jax.ShapeDtypeStruct((N, 4 * DOUT), _f32),
    )


_MESH = plsc.VectorSubcoreMesh(core_axis_name="c", subcore_axis_name="s")
_SC_PARAMS = pltpu.CompilerParams(use_tc_tiling_on_sc=False)
_SC_PARAMS_NL = pltpu.CompilerParams(use_tc_tiling_on_sc=False,
                                     needs_layout_passes=False)
_Z16 = lambda: jnp.zeros((16,), _f32)


def _cnt_kernel(didx_hbm, cnt_hbm, hist, ibuf, sl0, sl1):
    c = lax.axis_index("c")
    s = lax.axis_index("s")
    lo = s * CPT
    base_row = c * (IR // NC)
    sem_ld = (sl0, sl1)

    def zfill(i, _):
        hist[pl.ds(i * 16, 16)] = _Z16()
        return _

    lax.fori_loop(0, CPT // 16, zfill, None)
    ones = jnp.ones((16,), _f32)

    def ld(k, slot):
        pltpu.async_copy(didx_hbm.at[pl.ds(base_row + k * CCH, CCH)],
                         ibuf.at[slot], sem_ld[slot])

    def wait_ld(slot):
        pltpu.make_async_copy(didx_hbm.at[pl.ds(0, CCH)], ibuf.at[slot],
                              sem_ld[slot]).wait()

    ld(0, 0)
    ld(1, 1)

    def body(g, _):
        for u in range(2):
            wait_ld(u)

            def comp(r, _):
                for t in range(8):
                    d = ibuf[u, r, pl.ds(t * 16, 16)]
                    local = d - lo
                    mask = (local >= 0) & (local < CPT)
                    plsc.addupdate_scatter(hist, [local], ones, mask=mask)
                return _

            lax.fori_loop(0, CCH, comp, None)

            @pl.when(g < CNCH // 2 - 1)
            def _l():
                ld(2 * g + u + 2, u)
        return _

    lax.fori_loop(0, CNCH // 2, body, None)
    pltpu.sync_copy(hist, cnt_hbm.at[pl.ds(c * CNT + s * CPT, CPT)])


_cnt_call = pl.kernel(
    _cnt_kernel,
    out_type=jax.ShapeDtypeStruct((NC * CNT,), _f32),
    mesh=_MESH,
    compiler_params=_SC_PARAMS_NL,
    scratch_types=[
        pltpu.VMEM((CPT,), _f32),
        pltpu.VMEM((2, CCH, 128), jnp.int32),
        pltpu.SemaphoreType.DMA,
        pltpu.SemaphoreType.DMA,
    ],
)


def _scale_kernel(didx_hbm, inv_hbm, scl_hbm, idxv, sbuf, sem):
    c = lax.axis_index("c")
    s = lax.axis_index("s")
    wrow = (c * NS + s) * RW

    def body(g, _):
        rbase = wrow + g * 8
        pltpu.sync_copy(didx_hbm.at[pl.ds(rbase, 8)], idxv)
        for j in range(8):
            pltpu.async_copy(inv_hbm.at[idxv.at[j]],
                             sbuf.at[pl.ds(j * 128, 128)], sem).wait()
        ebase = rbase * 128

        def mask(k, _):
            pos = ebase + k * 16 + lax.iota(jnp.int32, 16)
            v = sbuf[pl.ds(k * 16, 16)]
            sbuf[pl.ds(k * 16, 16)] = jnp.where(pos < E, v, 0.0)
            return _

        lax.fori_loop(0, 64, mask, None)
        pltpu.sync_copy(sbuf, scl_hbm.at[pl.ds(ebase, 1024)])
        return _

    lax.fori_loop(0, RW // 8, body, None)


_scale_call = pl.kernel(
    _scale_kernel,
    out_type=jax.ShapeDtypeStruct((EP,), _f32),
    mesh=_MESH,
    compiler_params=_SC_PARAMS,
    scratch_types=[
        pltpu.VMEM((8, 128), jnp.int32),
        pltpu.VMEM((1024,), _f32),
        pltpu.SemaphoreType.DMA,
    ],
)


def _edge_kernel(tbl_hbm, pk_hbm, scl_hbm, p_hbm,
                 acc, rows, pkv, sclv,
                 sl0, sl1, sg0, sg1, sg2, ss0, ss1):
    c = lax.axis_index("c")
    s = lax.axis_index("s")
    wid = c * NS + s
    sem_ld = (sl0, sl1)
    sem_g = (sg0, sg1, sg2)
    sem_s = (ss0, ss1)

    def zfill(i, _):
        rows[0, i, pl.ds(0, 16)] = _Z16()
        rows[0, i, pl.ds(16, 16)] = _Z16()
        return _

    lax.fori_loop(0, BLK, zfill, None)
    for k in range(12):
        pltpu.async_copy(rows.at[0], acc.at[pl.ds(s * TPT + k * BLK, BLK)], sg0)
    pltpu.async_copy(rows.at[0, pl.ds(0, TPT - 12 * BLK)],
                     acc.at[pl.ds(s * TPT + 12 * BLK, TPT - 12 * BLK)], sg0)
    for k in range(12):
        pltpu.make_async_copy(rows.at[0], acc.at[pl.ds(0, BLK)], sg0).wait()
    pltpu.make_async_copy(rows.at[0, pl.ds(0, TPT - 12 * BLK)],
                          acc.at[pl.ds(0, TPT - 12 * BLK)], sg0).wait()
    plsc.subcore_barrier()

    wrow = wid * RW

    def ld(v, slot2, slot4, sem):
        rb = wrow + v * BR
        pltpu.async_copy(pk_hbm.at[pl.ds(rb, BR)], pkv.at[slot4], sem)
        pltpu.async_copy(scl_hbm.at[pl.ds(rb * 128, BLK)], sclv.at[slot2], sem)

    def wait_ld(slot2, slot4, sem):
        pltpu.make_async_copy(pk_hbm.at[pl.ds(0, BR)], pkv.at[slot4],
                              sem).wait()
        pltpu.make_async_copy(scl_hbm.at[pl.ds(0, BLK)], sclv.at[slot2],
                              sem).wait()

    def gath(slot4, slot3, sem):
        for j in range(BR):
            pltpu.async_copy(tbl_hbm.at[pkv.at[slot4, j, 0]],
                             rows.at[slot3, pl.ds(j * 128, 128)], sem)

    def wait_gath(slot3, sem):
        for j in range(BR):
            pltpu.make_async_copy(tbl_hbm.at[pkv.at[0, 0, 0]],
                                  rows.at[slot3, pl.ds(j * 128, 128)],
                                  sem).wait()

    def scat(slot3, slot4, sem):
        for j in range(BR):
            pltpu.async_copy(rows.at[slot3, pl.ds(j * 128, 128)],
                             acc.at[pkv.at[slot4, j, 1]], sem, add=True)

    def wait_scat(slot3, sem):
        for j in range(BR):
            pltpu.make_async_copy(rows.at[slot3, pl.ds(j * 128, 128)],
                                  acc.at[pkv.at[0, 0, 1]], sem).wait()

    ld(0, 0, 0, sem_ld[0])
    ld(1, 1, 1, sem_ld[1])
    wait_ld(0, 0, sem_ld[0])
    gath(0, 0, sem_g[0])

    def body(g, _):
        for u in range(12):
            v = g * 12 + u
            if u < 2:
                @pl.when(g > 0)
                def _a():
                    wait_scat((u + 1) % 3, sem_s[u % 2])
            else:
                wait_scat((u + 1) % 3, sem_s[u % 2])
            if u == 11:
                @pl.when(g < NBLK // 12 - 1)
                def _bc():
                    wait_ld((u + 1) % 2, (u + 1) % 4, sem_ld[(u + 1) % 2])
                    gath((u + 1) % 4, (u + 1) % 3, sem_g[(u + 1) % 3])
            else:
                wait_ld((u + 1) % 2, (u + 1) % 4, sem_ld[(u + 1) % 2])
                gath((u + 1) % 4, (u + 1) % 3, sem_g[(u + 1) % 3])
            wait_gath(u % 3, sem_g[u % 3])

            def smul(i2, _):
                vv = sclv[u % 2, pl.ds(i2 * 16, 16)]
                for t in range(16):
                    jj = i2 * 16 + t
                    sc = vv[t]
                    rows[u % 3, jj, pl.ds(0, 16)] = \
                        rows[u % 3, jj, pl.ds(0, 16)] * sc
                    rows[u % 3, jj, pl.ds(16, 16)] = \
                        rows[u % 3, jj, pl.ds(16, 16)] * sc
                return _

            lax.fori_loop(0, BLK // 16, smul, None)
            scat(u % 3, u % 4, sem_s[u % 2])
            if u < 10:
                ld(jnp.minimum(v + 2, NBLK - 1), u % 2, (u + 2) % 4,
                   sem_ld[u % 2])
            else:
                @pl.when(g < NBLK // 12 - 1)
                def _d():
                    ld(v + 2, u % 2, (u + 2) % 4, sem_ld[u % 2])
        return _

    lax.fori_loop(0, NBLK // 12, body, None)
    wait_scat((10 + 1) % 3, sem_s[0])
    wait_scat((11 + 1) % 3, sem_s[1])
    plsc.subcore_barrier()
    pltpu.sync_copy(acc.at[pl.ds(s * TPT, TPT)],
                    p_hbm.at[pl.ds(c * ACC + s * TPT, TPT)])


_edge_call = pl.kernel(
    _edge_kernel,
    out_type=jax.ShapeDtypeStruct((NC * ACC, DOUT), _f32),
    mesh=_MESH,
    compiler_params=_SC_PARAMS,
    scratch_types=[
        pltpu.VMEM_SHARED((ACC, DOUT), _f32),
        pltpu.VMEM((3, BLK, DOUT), _f32),
        pltpu.VMEM((4, BR, 2, 128), jnp.int32),
        pltpu.VMEM((2, BLK), _f32),
        pltpu.SemaphoreType.DMA,
        pltpu.SemaphoreType.DMA,
        pltpu.SemaphoreType.DMA,
        pltpu.SemaphoreType.DMA,
        pltpu.SemaphoreType.DMA,
        pltpu.SemaphoreType.DMA,
        pltpu.SemaphoreType.DMA,
    ],
)


def kernel(x, edge_index, edge_type, Wf, bf, W_rel, W_root, b_conv, ln_g, ln_b):
    src, dst = edge_index[0], edge_index[1]
    npad = EP - E
    pidx = jnp.arange(npad, dtype=jnp.int32) % 128
    src2 = jnp.concatenate([src, pidx]).reshape(IR, 128)
    dst2 = jnp.concatenate([dst, pidx + N]).reshape(IR, 128)
    et2 = jnp.concatenate([edge_type, jnp.zeros((npad,), jnp.int32)]).reshape(IR, 128)

    pk2, didx2 = _make_idx()(src2, et2, dst2)
    cnt = _cnt_call(didx2)
    inv = _make_inv()(cnt.reshape(2, CNT // 128, 128)).reshape(CNT)
    scale = _scale_call(didx2, inv)

    w_all = [jnp.concatenate([W_rel[l, 0], W_rel[l, 1], W_rel[l, 2], W_root[l]],
                             axis=1) for l in range(L)]
    vec = lambda a: a.reshape(1, DOUT)
    h4 = _make_pre()(x, Wf, vec(bf), w_all[0])
    for l in range(L):
        part = _edge_call(h4.reshape(4 * N, DOUT), pk2, scale)
        part = part.reshape(NC, ACC, DOUT)
        if l + 1 < L:
            h4 = _make_ln(False)(part, h4, vec(b_conv[l]), vec(ln_g),
                                 vec(ln_b), w_all[l + 1])
        else:
            out = _make_ln(True)(part, h4, vec(b_conv[l]), vec(ln_g), vec(ln_b))
    return out


# speedup vs baseline: 1.2694x; 1.2694x over previous
import jax
import jax.numpy as jnp
from jax import lax
from jax.experimental import pallas as pl
from jax.experimental.pallas import tpu as pltpu
from jax.experimental.pallas import tpu_sc as plsc

N = 50000
E = 1600000
DIN = 128
DOUT = 32
R = 3
L = 3

NC, NS = 2, 16
NW = NC * NS
NBLK = 204
BR = 2
BLK = BR * 128
RW = NBLK * BR
IR = NW * RW
EP = IR * 128
CBR = 6
CBLK = 68
ACC = 50176
TPT = ACC // NS
CNT = 4 * ACC
CPT = CNT // NS
BN = 2000

_f32 = jnp.float32


def _idx_body(src_ref, et_ref, dst_ref, gidx_ref, didx_ref):
    et = et_ref[...]
    gidx_ref[...] = src_ref[...] * 4 + et
    didx_ref[...] = dst_ref[...] * 4 + et


def _make_idx():
    blk = pl.BlockSpec((544, 128), lambda i: (i, 0))
    return pl.pallas_call(
        _idx_body,
        grid=(IR // 544,),
        in_specs=[blk, blk, blk],
        out_specs=[blk, blk],
        out_shape=[jax.ShapeDtypeStruct((IR, 128), jnp.int32)] * 2,
    )


def _inv_body(cnt_ref, inv_ref):
    c = cnt_ref[0] + cnt_ref[1]
    inv_ref[...] = 1.0 / jnp.maximum(c, 1.0)


def _make_inv():
    return pl.pallas_call(
        _inv_body,
        in_specs=[pl.BlockSpec((2, CNT // 128, 128), lambda: (0, 0, 0))],
        out_specs=pl.BlockSpec((CNT // 128, 128), lambda: (0, 0)),
        out_shape=jax.ShapeDtypeStruct((CNT // 128, 128), _f32),
    )


def _pre_body(x_ref, wf_ref, bf_ref, wa_ref, h4_ref):
    h0 = jnp.dot(x_ref[...], wf_ref[...], preferred_element_type=_f32)
    h0 = h0 + bf_ref[...]
    h4_ref[...] = jnp.dot(h0, wa_ref[...], preferred_element_type=_f32)


def _make_pre():
    return pl.pallas_call(
        _pre_body,
        grid=(N // BN,),
        in_specs=[
            pl.BlockSpec((BN, DIN), lambda i: (i, 0)),
            pl.BlockSpec((DIN, DOUT), lambda i: (0, 0)),
            pl.BlockSpec((1, DOUT), lambda i: (0, 0)),
            pl.BlockSpec((DOUT, 4 * DOUT), lambda i: (0, 0)),
        ],
        out_specs=pl.BlockSpec((BN, 4 * DOUT), lambda i: (i, 0)),
        out_shape=jax.ShapeDtypeStruct((N, 4 * DOUT), _f32),
    )


def _ln_core(p_ref, root_ref, bc_ref, g_ref, b_ref):
    y = p_ref[0] + p_ref[1] + root_ref[...][:, 96:128] + bc_ref[...]
    y = jnp.maximum(y, 0.0)
    mu = jnp.mean(y, axis=-1, keepdims=True)
    d = y - mu
    var = jnp.mean(d * d, axis=-1, keepdims=True)
    return d * lax.rsqrt(var + 1e-5) * g_ref[...] + b_ref[...]


def _ln_mid_body(p_ref, root_ref, bc_ref, g_ref, b_ref, wn_ref, out_ref):
    yn = _ln_core(p_ref, root_ref, bc_ref, g_ref, b_ref)
    out_ref[...] = jnp.dot(yn, wn_ref[...], preferred_element_type=_f32)


def _ln_last_body(p_ref, root_ref, bc_ref, g_ref, b_ref, out_ref):
    out_ref[...] = _ln_core(p_ref, root_ref, bc_ref, g_ref, b_ref)


def _make_ln(last):
    vec = pl.BlockSpec((1, DOUT), lambda i: (0, 0))
    in_specs = [
        pl.BlockSpec((2, BN, DOUT), lambda i: (0, i, 0)),
        pl.BlockSpec((BN, 4 * DOUT), lambda i: (i, 0)),
        vec, vec, vec,
    ]
    if last:
        return pl.pallas_call(
            _ln_last_body,
            grid=(N // BN,),
            in_specs=in_specs,
            out_specs=pl.BlockSpec((BN, DOUT), lambda i: (i, 0)),
            out_shape=jax.ShapeDtypeStruct((N, DOUT), _f32),
        )
    in_specs = in_specs + [pl.BlockSpec((DOUT, 4 * DOUT), lambda i: (0, 0))]
    return pl.pallas_call(
        _ln_mid_body,
        grid=(N // BN,),
        in_specs=in_specs,
        out_specs=pl.BlockSpec((BN, 4 * DOUT), lambda i: (i, 0)),
        out_shape=jax.ShapeDtypeStruct((N, 4 * DOUT), _f32),
    )


_MESH = plsc.VectorSubcoreMesh(core_axis_name="c", subcore_axis_name="s")
_SC_PARAMS = pltpu.CompilerParams(use_tc_tiling_on_sc=False)
_Z16 = lambda: jnp.zeros((16,), _f32)


def _cnt_kernel(didx_hbm, cnt_hbm, acc, zbuf, ones, idxv,
                sl0, sl1, ss0, ss1):
    c = lax.axis_index("c")
    s = lax.axis_index("s")
    wid = c * NS + s
    sem_ld = (sl0, sl1)
    sem_sc = (ss0, ss1)

    def zfill(i, _):
        zbuf[pl.ds(i * 16, 16)] = _Z16()
        return _

    lax.fori_loop(0, CPT // 16, zfill, None)

    def ofill(i, _):
        ones[pl.ds(i * 16, 16)] = jnp.ones((16,), _f32)
        return _

    lax.fori_loop(0, 8, ofill, None)
    pltpu.sync_copy(zbuf, acc.at[pl.ds(s * CPT, CPT)])
    plsc.subcore_barrier()

    wrow = wid * CBLK * CBR

    def ld(v, slot, sem):
        pltpu.async_copy(didx_hbm.at[pl.ds(wrow + v * CBR, CBR)],
                         idxv.at[slot], sem)

    def wait_ld(slot, sem):
        pltpu.make_async_copy(didx_hbm.at[pl.ds(0, CBR)], idxv.at[slot],
                              sem).wait()

    def scat(slot, sem):
        for j in range(CBR):
            pltpu.async_copy(ones, acc.at[idxv.at[slot, j]], sem, add=True)

    def wait_scat(sem):
        for _ in range(CBR):
            pltpu.make_async_copy(ones, acc.at[idxv.at[0, 0]], sem).wait()

    ld(0, 0, sem_ld[0])
    ld(1, 1, sem_ld[1])

    def body(g, _):
        for u in range(4):
            v = g * 4 + u
            if u < 2:
                @pl.when(g > 0)
                def _w():
                    wait_scat(sem_sc[u % 2])
            else:
                wait_scat(sem_sc[u % 2])
            wait_ld(u, sem_ld[u % 2])
            scat(u, sem_sc[u % 2])
            if u < 2:
                ld(jnp.minimum(v + 2, CBLK - 1), (u + 2) % 4, sem_ld[u % 2])
            else:
                @pl.when(g < CBLK // 4 - 1)
                def _l():
                    ld(v + 2, (u + 2) % 4, sem_ld[u % 2])
        return _

    lax.fori_loop(0, CBLK // 4, body, None)
    wait_scat(sem_sc[0])
    wait_scat(sem_sc[1])
    plsc.subcore_barrier()
    pltpu.sync_copy(acc.at[pl.ds(s * CPT, CPT)],
                    cnt_hbm.at[pl.ds(c * CNT + s * CPT, CPT)])


_cnt_call = pl.kernel(
    _cnt_kernel,
    out_type=jax.ShapeDtypeStruct((NC * CNT,), _f32),
    mesh=_MESH,
    compiler_params=_SC_PARAMS,
    scratch_types=[
        pltpu.VMEM_SHARED((CNT,), _f32),
        pltpu.VMEM((CPT,), _f32),
        pltpu.VMEM((128,), _f32),
        pltpu.VMEM((4, CBR, 128), jnp.int32),
        pltpu.SemaphoreType.DMA,
        pltpu.SemaphoreType.DMA,
        pltpu.SemaphoreType.DMA,
        pltpu.SemaphoreType.DMA,
    ],
)


def _scale_kernel(didx_hbm, inv_hbm, scl_hbm, idxv, sbuf, sem):
    c = lax.axis_index("c")
    s = lax.axis_index("s")
    wrow = (c * NS + s) * RW

    def body(g, _):
        rbase = wrow + g * 8
        pltpu.sync_copy(didx_hbm.at[pl.ds(rbase, 8)], idxv)
        for j in range(8):
            pltpu.async_copy(inv_hbm.at[idxv.at[j]],
                             sbuf.at[pl.ds(j * 128, 128)], sem).wait()
        ebase = rbase * 128

        def mask(k, _):
            pos = ebase + k * 16 + lax.iota(jnp.int32, 16)
            v = sbuf[pl.ds(k * 16, 16)]
            sbuf[pl.ds(k * 16, 16)] = jnp.where(pos < E, v, 0.0)
            return _

        lax.fori_loop(0, 64, mask, None)
        pltpu.sync_copy(sbuf, scl_hbm.at[pl.ds(ebase, 1024)])
        return _

    lax.fori_loop(0, RW // 8, body, None)


_scale_call = pl.kernel(
    _scale_kernel,
    out_type=jax.ShapeDtypeStruct((EP,), _f32),
    mesh=_MESH,
    compiler_params=_SC_PARAMS,
    scratch_types=[
        pltpu.VMEM((8, 128), jnp.int32),
        pltpu.VMEM((1024,), _f32),
        pltpu.SemaphoreType.DMA,
    ],
)


def _edge_kernel(tbl_hbm, gidx_hbm, dst_hbm, scl_hbm, p_hbm,
                 acc, rows, gixv, dstv, sclv,
                 sl0, sl1, sg0, sg1, sg2, ss0, ss1):
    c = lax.axis_index("c")
    s = lax.axis_index("s")
    wid = c * NS + s
    sem_ld = (sl0, sl1)
    sem_g = (sg0, sg1, sg2)
    sem_s = (ss0, ss1)

    def zfill(i, _):
        rows[0, i, pl.ds(0, 16)] = _Z16()
        rows[0, i, pl.ds(16, 16)] = _Z16()
        return _

    lax.fori_loop(0, BLK, zfill, None)
    for k in range(12):
        pltpu.async_copy(rows.at[0], acc.at[pl.ds(s * TPT + k * BLK, BLK)], sg0)
    pltpu.async_copy(rows.at[0, pl.ds(0, TPT - 12 * BLK)],
                     acc.at[pl.ds(s * TPT + 12 * BLK, TPT - 12 * BLK)], sg0)
    for k in range(12):
        pltpu.make_async_copy(rows.at[0], acc.at[pl.ds(0, BLK)], sg0).wait()
    pltpu.make_async_copy(rows.at[0, pl.ds(0, TPT - 12 * BLK)],
                          acc.at[pl.ds(0, TPT - 12 * BLK)], sg0).wait()
    plsc.subcore_barrier()

    wrow = wid * RW

    def ld(v, slot2, slot4, sem):
        rb = wrow + v * BR
        pltpu.async_copy(gidx_hbm.at[pl.ds(rb, BR)], gixv.at[slot2], sem)
        pltpu.async_copy(scl_hbm.at[pl.ds(rb * 128, BLK)], sclv.at[slot2], sem)
        pltpu.async_copy(dst_hbm.at[pl.ds(rb, BR)], dstv.at[slot4], sem)

    def wait_ld(slot2, slot4, sem):
        pltpu.make_async_copy(gidx_hbm.at[pl.ds(0, BR)], gixv.at[slot2],
                              sem).wait()
        pltpu.make_async_copy(scl_hbm.at[pl.ds(0, BLK)], sclv.at[slot2],
                              sem).wait()
        pltpu.make_async_copy(dst_hbm.at[pl.ds(0, BR)], dstv.at[slot4],
                              sem).wait()

    def gath(slot2, slot3, sem):
        for j in range(BR):
            pltpu.async_copy(tbl_hbm.at[gixv.at[slot2, j]],
                             rows.at[slot3, pl.ds(j * 128, 128)], sem)

    def wait_gath(slot3, sem):
        for j in range(BR):
            pltpu.make_async_copy(tbl_hbm.at[gixv.at[0, 0]],
                                  rows.at[slot3, pl.ds(j * 128, 128)],
                                  sem).wait()

    def scat(slot3, slot4, sem):
        for j in range(BR):
            pltpu.async_copy(rows.at[slot3, pl.ds(j * 128, 128)],
                             acc.at[dstv.at[slot4, j]], sem, add=True)

    def wait_scat(slot3, sem):
        for j in range(BR):
            pltpu.make_async_copy(rows.at[slot3, pl.ds(j * 128, 128)],
                                  acc.at[dstv.at[0, 0]], sem).wait()

    ld(0, 0, 0, sem_ld[0])
    ld(1, 1, 1, sem_ld[1])
    wait_ld(0, 0, sem_ld[0])
    gath(0, 0, sem_g[0])

    def body(g, _):
        for u in range(12):
            v = g * 12 + u
            if u < 2:
                @pl.when(g > 0)
                def _a():
                    wait_scat((u + 1) % 3, sem_s[u % 2])
            else:
                wait_scat((u + 1) % 3, sem_s[u % 2])
            if u == 11:
                @pl.when(g < NBLK // 12 - 1)
                def _bc():
                    wait_ld((u + 1) % 2, (u + 1) % 4, sem_ld[(u + 1) % 2])
                    gath((u + 1) % 2, (u + 1) % 3, sem_g[(u + 1) % 3])
            else:
                wait_ld((u + 1) % 2, (u + 1) % 4, sem_ld[(u + 1) % 2])
                gath((u + 1) % 2, (u + 1) % 3, sem_g[(u + 1) % 3])
            wait_gath(u % 3, sem_g[u % 3])

            def smul(i2, _):
                vv = sclv[u % 2, pl.ds(i2 * 16, 16)]
                for t in range(16):
                    jj = i2 * 16 + t
                    sc = vv[t]
                    rows[u % 3, jj, pl.ds(0, 16)] = \
                        rows[u % 3, jj, pl.ds(0, 16)] * sc
                    rows[u % 3, jj, pl.ds(16, 16)] = \
                        rows[u % 3, jj, pl.ds(16, 16)] * sc
                return _

            lax.fori_loop(0, BLK // 16, smul, None)
            scat(u % 3, u % 4, sem_s[u % 2])
            if u < 10:
                ld(jnp.minimum(v + 2, NBLK - 1), u % 2, (u + 2) % 4,
                   sem_ld[u % 2])
            else:
                @pl.when(g < NBLK // 12 - 1)
                def _d():
                    ld(v + 2, u % 2, (u + 2) % 4, sem_ld[u % 2])
        return _

    lax.fori_loop(0, NBLK // 12, body, None)
    wait_scat((10 + 1) % 3, sem_s[0])
    wait_scat((11 + 1) % 3, sem_s[1])
    plsc.subcore_barrier()
    pltpu.sync_copy(acc.at[pl.ds(s * TPT, TPT)],
                    p_hbm.at[pl.ds(c * ACC + s * TPT, TPT)])


_edge_call = pl.kernel(
    _edge_kernel,
    out_type=jax.ShapeDtypeStruct((NC * ACC, DOUT), _f32),
    mesh=_MESH,
    compiler_params=_SC_PARAMS,
    scratch_types=[
        pltpu.VMEM_SHARED((ACC, DOUT), _f32),
        pltpu.VMEM((3, BLK, DOUT), _f32),
        pltpu.VMEM((2, BR, 128), jnp.int32),
        pltpu.VMEM((4, BR, 128), jnp.int32),
        pltpu.VMEM((2, BLK), _f32),
        pltpu.SemaphoreType.DMA,
        pltpu.SemaphoreType.DMA,
        pltpu.SemaphoreType.DMA,
        pltpu.SemaphoreType.DMA,
        pltpu.SemaphoreType.DMA,
        pltpu.SemaphoreType.DMA,
        pltpu.SemaphoreType.DMA,
    ],
)


def kernel(x, edge_index, edge_type, Wf, bf, W_rel, W_root, b_conv, ln_g, ln_b):
    src, dst = edge_index[0], edge_index[1]
    npad = EP - E
    pidx = jnp.arange(npad, dtype=jnp.int32) % 128
    src2 = jnp.concatenate([src, pidx]).reshape(IR, 128)
    dst2 = jnp.concatenate([dst, pidx + N]).reshape(IR, 128)
    et2 = jnp.concatenate([edge_type, jnp.zeros((npad,), jnp.int32)]).reshape(IR, 128)

    gidx2, didx2 = _make_idx()(src2, et2, dst2)
    cnt = _cnt_call(didx2)
    inv = _make_inv()(cnt.reshape(2, CNT // 128, 128)).reshape(CNT)
    scale = _scale_call(didx2, inv)

    w_all = [jnp.concatenate([W_rel[l, 0], W_rel[l, 1], W_rel[l, 2], W_root[l]],
                             axis=1) for l in range(L)]
    vec = lambda a: a.reshape(1, DOUT)
    h4 = _make_pre()(x, Wf, vec(bf), w_all[0])
    for l in range(L):
        part = _edge_call(h4.reshape(4 * N, DOUT), gidx2, dst2, scale)
        part = part.reshape(NC, ACC, DOUT)
        if l + 1 < L:
            h4 = _make_ln(False)(part, h4, vec(b_conv[l]), vec(ln_g),
                                 vec(ln_b), w_all[l + 1])
        else:
            out = _make_ln(True)(part, h4, vec(b_conv[l]), vec(ln_g), vec(ln_b))
    return out
